# baseline (speedup 1.0000x reference)
import jax
import jax.numpy as jnp
from jax import lax
from jax.experimental import pallas as pl
from jax.experimental.pallas import tpu as pltpu
from jax.experimental.pallas import tpu_sc as plsc

NUM_CORES = 2
NUM_SUBCORES = 16
NW = NUM_CORES * NUM_SUBCORES
CHUNK = 128


def _emb_body(x_hbm, w_hbm, out_hbm, idx_v, buf_a, buf_b, gsem_a, gsem_b):
    nch = x_hbm.shape[1]
    wid = lax.axis_index("s") * NUM_CORES + lax.axis_index("c")
    pltpu.sync_copy(x_hbm.at[wid], idx_v)
    pltpu.async_copy(w_hbm.at[idx_v.at[0]], buf_a, gsem_a)

    def body(i, carry):
        j = 2 * i
        pltpu.async_copy(w_hbm.at[idx_v.at[j + 1]], buf_b, gsem_b)
        pltpu.make_async_copy(w_hbm.at[idx_v.at[j]], buf_a, gsem_a).wait()
        pltpu.sync_copy(buf_a, out_hbm.at[wid, j])

        @pl.when(j + 2 < nch)
        def _():
            pltpu.async_copy(w_hbm.at[idx_v.at[j + 2]], buf_a, gsem_a)

        pltpu.make_async_copy(w_hbm.at[idx_v.at[j + 1]], buf_b, gsem_b).wait()
        pltpu.sync_copy(buf_b, out_hbm.at[wid, j + 1])
        return carry

    lax.fori_loop(0, nch // 2, body, 0)


def kernel(x, W):
    orig_shape = x.shape
    d = W.shape[1]
    b = x.size
    group = NW * CHUNK * 2
    b_pad = ((b + group - 1) // group) * group
    x_flat = x.reshape(-1).astype(jnp.int32)
    if b_pad != b:
        x_flat = jnp.pad(x_flat, (0, b_pad - b))
    nch = b_pad // (NW * CHUNK)
    x_r = x_flat.reshape(NW, nch, CHUNK)

    mesh = plsc.VectorSubcoreMesh(core_axis_name="c", subcore_axis_name="s")
    out = pl.kernel(
        _emb_body,
        out_type=jax.ShapeDtypeStruct((NW, nch, CHUNK, d), jnp.float32),
        mesh=mesh,
        scratch_types=[
            pltpu.VMEM((nch, CHUNK), jnp.int32),
            pltpu.VMEM((CHUNK, d), jnp.float32),
            pltpu.VMEM((CHUNK, d), jnp.float32),
            pltpu.SemaphoreType.DMA,
            pltpu.SemaphoreType.DMA,
        ],
        compiler_params=pltpu.CompilerParams(use_tc_tiling_on_sc=False),
    )(x_r, W)

    out = out.reshape(-1, d)
    if b_pad != b:
        out = out[:b]
    return out.reshape(*orig_shape, d)


# speedup vs baseline: 1.0156x; 1.0156x over previous
import jax
import jax.numpy as jnp
from jax import lax
from jax.experimental import pallas as pl
from jax.experimental.pallas import tpu as pltpu
from jax.experimental.pallas import tpu_sc as plsc

NUM_CORES = 2
NUM_SUBCORES = 16
NW = NUM_CORES * NUM_SUBCORES
CHUNK = 512


def _emb_body(x_hbm, w_hbm, out_hbm, idx_v, buf_a, buf_b, gsem_a, gsem_b):
    nch = x_hbm.shape[1]
    wid = lax.axis_index("s") * NUM_CORES + lax.axis_index("c")
    pltpu.sync_copy(x_hbm.at[wid], idx_v)
    pltpu.async_copy(w_hbm.at[idx_v.at[0]], buf_a, gsem_a)

    def body(i, carry):
        j = 2 * i
        pltpu.async_copy(w_hbm.at[idx_v.at[j + 1]], buf_b, gsem_b)
        pltpu.make_async_copy(w_hbm.at[idx_v.at[j]], buf_a, gsem_a).wait()
        pltpu.sync_copy(buf_a, out_hbm.at[wid, j])

        @pl.when(j + 2 < nch)
        def _():
            pltpu.async_copy(w_hbm.at[idx_v.at[j + 2]], buf_a, gsem_a)

        pltpu.make_async_copy(w_hbm.at[idx_v.at[j + 1]], buf_b, gsem_b).wait()
        pltpu.sync_copy(buf_b, out_hbm.at[wid, j + 1])
        return carry

    lax.fori_loop(0, nch // 2, body, 0)


def kernel(x, W):
    orig_shape = x.shape
    d = W.shape[1]
    b = x.size
    group = NW * CHUNK * 2
    b_pad = ((b + group - 1) // group) * group
    x_flat = x.reshape(-1).astype(jnp.int32)
    if b_pad != b:
        x_flat = jnp.pad(x_flat, (0, b_pad - b))
    nch = b_pad // (NW * CHUNK)
    x_r = x_flat.reshape(NW, nch, CHUNK)

    mesh = plsc.VectorSubcoreMesh(core_axis_name="c", subcore_axis_name="s")
    out = pl.kernel(
        _emb_body,
        out_type=jax.ShapeDtypeStruct((NW, nch, CHUNK, d), jnp.float32),
        mesh=mesh,
        scratch_types=[
            pltpu.VMEM((nch, CHUNK), jnp.int32),
            pltpu.VMEM((CHUNK, d), jnp.float32),
            pltpu.VMEM((CHUNK, d), jnp.float32),
            pltpu.SemaphoreType.DMA,
            pltpu.SemaphoreType.DMA,
        ],
        compiler_params=pltpu.CompilerParams(use_tc_tiling_on_sc=False),
    )(x_r, W)

    out = out.reshape(-1, d)
    if b_pad != b:
        out = out[:b]
    return out.reshape(*orig_shape, d)
